# baseline (speedup 1.0000x reference)
import functools

import jax
import jax.numpy as jnp
from jax import lax
from jax.experimental import pallas as pl
from jax.experimental.pallas import tpu as pltpu
from jax.experimental.pallas import tpu_sc as plsc

N = 10000
D = 128
K = 10
ALPHA = 0.1
EPS = 1e-5
E = 320000

NC = 2
NS = 16
NW = NC * NS

NPAD = 10112
RPT = NPAD // NS
CHUNK = 128
ECH = 80
EPT = ECH * CHUNK
EPAD = EPT * NW
NBUF = 2

def _edge_body(u_hbm, src_hbm, dst_hbm, out_hbm, s_sh, src_v, dst_v, rowbuf,
               zbuf, zsem, sem):
    cid = lax.axis_index("c")
    sid = lax.axis_index("s")
    wid = cid * NS + sid
    for r in range(16):
        for j in range(D // 16):
            zbuf[r, pl.ds(j * 16, 16)] = jnp.zeros((16,), jnp.float32)
    zbase = sid * RPT
    def _z(i, c):
        pltpu.sync_copy(zbuf, s_sh.at[pl.ds(zbase + i * 16, 16)])
        return c
    lax.fori_loop(0, RPT // 16, _z, 0)
    pltpu.sync_copy(zbuf.at[pl.ds(0, 8)], s_sh.at[pl.ds(zbase + 624, 8)])
    pltpu.sync_copy(src_hbm.at[wid], src_v)
    pltpu.sync_copy(dst_hbm.at[wid], dst_v)
    plsc.subcore_barrier()
    def _e(c, carry):
        pltpu.async_copy(u_hbm.at[src_v.at[c]], rowbuf, sem).wait()
        pltpu.sync_copy(rowbuf, s_sh.at[dst_v.at[c]], add=True)
        return carry
    lax.fori_loop(0, ECH, _e, 0)
    plsc.subcore_barrier()
    pltpu.sync_copy(s_sh.at[pl.ds(zbase, RPT)],
                    out_hbm.at[cid].at[pl.ds(zbase, RPT)])


def _edge_call(u, srcb, dstb):
    mesh = plsc.VectorSubcoreMesh(core_axis_name="c", subcore_axis_name="s")
    f = pl.kernel(
        _edge_body,
        out_type=jax.ShapeDtypeStruct((NC, NPAD, D), jnp.float32),
        mesh=mesh,
        scratch_types=[
            pltpu.VMEM_SHARED((NPAD, D), jnp.float32),
            pltpu.VMEM((ECH, CHUNK), jnp.int32),
            pltpu.VMEM((ECH, CHUNK), jnp.int32),
            pltpu.VMEM((CHUNK, D), jnp.float32),
            pltpu.VMEM((16, D), jnp.float32),
            pltpu.SemaphoreType.DMA,
            pltpu.SemaphoreType.DMA,
        ],
    )
    return f(u, srcb, dstb)


def _gelu(x):
    return 0.5 * x * (1.0 + lax.erf(x * 0.7071067811865476))


def _ln(h, g, b):
    mu = jnp.mean(h, axis=-1, keepdims=True)
    var = jnp.mean((h - mu) ** 2, axis=-1, keepdims=True)
    return (h - mu) * jax.lax.rsqrt(var + EPS) * g + b


def _mlp_body(x_ref, w0_ref, b0_ref, g0_ref, bt0_ref, w1_ref, b1_ref, g1_ref,
              bt1_ref, w2_ref, b2_ref, degp_ref, h0_ref, u0_ref, dinv_ref):
    x = x_ref[...]
    h = jnp.dot(x, w0_ref[...], preferred_element_type=jnp.float32) + b0_ref[...]
    h = _gelu(h)
    h = _ln(h, g0_ref[...], bt0_ref[...])
    h = jnp.dot(h, w1_ref[...], preferred_element_type=jnp.float32) + b1_ref[...]
    h = _gelu(h)
    h = _ln(h, g1_ref[...], bt1_ref[...])
    h = jnp.dot(h, w2_ref[...], preferred_element_type=jnp.float32) + b2_ref[...]
    degp = degp_ref[...]
    deg = degp[0, :, 0] + degp[1, :, 0] + 1.0
    dinv = jax.lax.rsqrt(deg)[:, None]
    rows = lax.broadcasted_iota(jnp.int32, (NPAD, 1), 0)
    h = jnp.where(rows < N, h, 0.0)
    h0_ref[...] = h
    u0_ref[...] = h * dinv
    dinv_ref[...] = jnp.broadcast_to(dinv, (NPAD, D))


def _mlp_call(xpad, W0, b0, g0, bt0, W1, b1, g1, bt1, W2, b2, degp):
    out = [jax.ShapeDtypeStruct((NPAD, D), jnp.float32)] * 3
    return pl.pallas_call(_mlp_body, out_shape=out)(
        xpad, W0, b0, g0, bt0, W1, b1, g1, bt1, W2, b2, degp)


def _upd_body(s_ref, u_ref, u0_ref, dinv_ref, out_ref):
    s = s_ref[0] + s_ref[1] + u_ref[...]
    di = dinv_ref[...]
    out_ref[...] = (1.0 - ALPHA) * di * di * s + ALPHA * u0_ref[...]


def _upd_call(S, u, u0, dinv):
    return pl.pallas_call(
        _upd_body, out_shape=jax.ShapeDtypeStruct((NPAD, D), jnp.float32)
    )(S, u, u0, dinv)


def _fin_body(s_ref, u_ref, h0_ref, dinv_ref, out_ref):
    s = s_ref[0] + s_ref[1] + u_ref[...]
    out_ref[...] = (1.0 - ALPHA) * dinv_ref[...] * s + ALPHA * h0_ref[...]


def _fin_call(S, u, h0, dinv):
    return pl.pallas_call(
        _fin_body, out_shape=jax.ShapeDtypeStruct((NPAD, D), jnp.float32)
    )(S, u, h0, dinv)


def kernel(x, edge_index, W0, b0, g0, bt0, W1, b1, g1, bt1, W2, b2):
    src = edge_index[0]
    dst = edge_index[1]
    fill = jnp.full((EPAD - E,), N, jnp.int32)
    srcb = jnp.concatenate([src, fill]).reshape(NW, ECH, CHUNK)
    dstb = jnp.concatenate([dst, fill]).reshape(NW, ECH, CHUNK)
    xpad = jnp.pad(x, ((0, NPAD - N), (0, 0)))

    ones = jnp.ones((NPAD, D), jnp.float32)
    degp = _edge_call(ones, srcb, dstb)
    h0, u0, dinv = _mlp_call(xpad, W0, b0, g0, bt0, W1, b1, g1, bt1, W2, b2,
                             degp)
    u = u0
    for k in range(K):
        S = _edge_call(u, srcb, dstb)
        if k < K - 1:
            u = _upd_call(S, u, u0, dinv)
        else:
            h = _fin_call(S, u, h0, dinv)
    return h[:N]


# speedup vs baseline: 1.6082x; 1.6082x over previous
import functools

import jax
import jax.numpy as jnp
from jax import lax
from jax.experimental import pallas as pl
from jax.experimental.pallas import tpu as pltpu
from jax.experimental.pallas import tpu_sc as plsc

N = 10000
D = 128
K = 10
ALPHA = 0.1
EPS = 1e-5
E = 320000

NC = 2
NS = 16
NW = NC * NS

NPAD = 10112
RPT = NPAD // NS
CHUNK = 128
ECH = 79
EPT = ECH * CHUNK
EPAD = EPT * NW
NBUF = 2

def _edge_body(u_hbm, src_hbm, dst_hbm, out_hbm, s_sh, src_v, dst_v, rowbuf,
               zbuf, zsem, sem):
    cid = lax.axis_index("c")
    sid = lax.axis_index("s")
    wid = cid * NS + sid
    for r in range(16):
        for j in range(D // 16):
            zbuf[r, pl.ds(j * 16, 16)] = jnp.zeros((16,), jnp.float32)
    zbase = sid * RPT
    def _z(i, c):
        pltpu.sync_copy(zbuf, s_sh.at[pl.ds(zbase + i * 16, 16)])
        return c
    lax.fori_loop(0, RPT // 16, _z, 0)
    pltpu.sync_copy(zbuf.at[pl.ds(0, 8)], s_sh.at[pl.ds(zbase + 624, 8)])
    pltpu.sync_copy(src_hbm.at[wid], src_v)
    pltpu.sync_copy(dst_hbm.at[wid], dst_v)
    plsc.subcore_barrier()
    def _e(c, carry):
        pltpu.async_copy(u_hbm.at[src_v.at[c]], rowbuf, sem).wait()
        pltpu.sync_copy(rowbuf, s_sh.at[dst_v.at[c]], add=True)
        return carry
    lax.fori_loop(0, ECH, _e, 0)
    plsc.subcore_barrier()
    pltpu.sync_copy(s_sh.at[pl.ds(zbase, RPT)],
                    out_hbm.at[cid].at[pl.ds(zbase, RPT)])


def _edge_call(u, srcb, dstb):
    mesh = plsc.VectorSubcoreMesh(core_axis_name="c", subcore_axis_name="s")
    f = pl.kernel(
        _edge_body,
        out_type=jax.ShapeDtypeStruct((NC, NPAD, D), jnp.float32),
        mesh=mesh,
        scratch_types=[
            pltpu.VMEM_SHARED((NPAD, D), jnp.float32),
            pltpu.VMEM((ECH, CHUNK), jnp.int32),
            pltpu.VMEM((ECH, CHUNK), jnp.int32),
            pltpu.VMEM((CHUNK, D), jnp.float32),
            pltpu.VMEM((16, D), jnp.float32),
            pltpu.SemaphoreType.DMA,
            pltpu.SemaphoreType.DMA,
        ],
    )
    return f(u, srcb, dstb)


def _gelu(x):
    return 0.5 * x * (1.0 + lax.erf(x * 0.7071067811865476))


def _ln(h, g, b):
    mu = jnp.mean(h, axis=-1, keepdims=True)
    var = jnp.mean((h - mu) ** 2, axis=-1, keepdims=True)
    return (h - mu) * jax.lax.rsqrt(var + EPS) * g + b


def _mlp_body(x_ref, w0_ref, b0_ref, g0_ref, bt0_ref, w1_ref, b1_ref, g1_ref,
              bt1_ref, w2_ref, b2_ref, degp_ref, h0_ref, u0_ref, dinv_ref):
    x = x_ref[...]
    h = jnp.dot(x, w0_ref[...], preferred_element_type=jnp.float32) + b0_ref[...]
    h = _gelu(h)
    h = _ln(h, g0_ref[...], bt0_ref[...])
    h = jnp.dot(h, w1_ref[...], preferred_element_type=jnp.float32) + b1_ref[...]
    h = _gelu(h)
    h = _ln(h, g1_ref[...], bt1_ref[...])
    h = jnp.dot(h, w2_ref[...], preferred_element_type=jnp.float32) + b2_ref[...]
    degp = degp_ref[...]
    deg = degp[0, :, 0] + degp[1, :, 0] + 1.0
    dinv = jax.lax.rsqrt(deg)[:, None]
    rows = lax.broadcasted_iota(jnp.int32, (NPAD, 1), 0)
    h = jnp.where(rows < N, h, 0.0)
    h0_ref[...] = h
    u0_ref[...] = h * dinv
    dinv_ref[...] = jnp.broadcast_to(dinv, (NPAD, D))


def _mlp_call(xpad, W0, b0, g0, bt0, W1, b1, g1, bt1, W2, b2, degp):
    out = [jax.ShapeDtypeStruct((NPAD, D), jnp.float32)] * 3
    return pl.pallas_call(_mlp_body, out_shape=out)(
        xpad, W0, b0, g0, bt0, W1, b1, g1, bt1, W2, b2, degp)


def _upd_body(s_ref, u_ref, u0_ref, dinv_ref, out_ref):
    s = s_ref[0] + s_ref[1] + u_ref[...]
    di = dinv_ref[...]
    out_ref[...] = (1.0 - ALPHA) * di * di * s + ALPHA * u0_ref[...]


def _upd_call(S, u, u0, dinv):
    return pl.pallas_call(
        _upd_body, out_shape=jax.ShapeDtypeStruct((NPAD, D), jnp.float32)
    )(S, u, u0, dinv)


def _fin_body(s_ref, u_ref, h0_ref, dinv_ref, out_ref):
    s = s_ref[0] + s_ref[1] + u_ref[...]
    out_ref[...] = (1.0 - ALPHA) * dinv_ref[...] * s + ALPHA * h0_ref[...]


def _fin_call(S, u, h0, dinv):
    return pl.pallas_call(
        _fin_body, out_shape=jax.ShapeDtypeStruct((NPAD, D), jnp.float32)
    )(S, u, h0, dinv)


def kernel(x, edge_index, W0, b0, g0, bt0, W1, b1, g1, bt1, W2, b2):
    src = edge_index[0]
    dst = edge_index[1]
    fill = jnp.full((EPAD - E,), N, jnp.int32)
    srcb = jnp.concatenate([src, fill]).reshape(NW, ECH, CHUNK)
    dstb = jnp.concatenate([dst, fill]).reshape(NW, ECH, CHUNK)
    xpad = jnp.pad(x, ((0, NPAD - N), (0, 0)))

    ones = jnp.ones((NPAD, D), jnp.float32)
    degp = _edge_call(ones, srcb, dstb)
    h0, u0, dinv = _mlp_call(xpad, W0, b0, g0, bt0, W1, b1, g1, bt1, W2, b2,
                             degp)
    u = u0
    for k in range(K):
        S = _edge_call(u, srcb, dstb)
        if k < K - 1:
            u = _upd_call(S, u, u0, dinv)
        else:
            h = _fin_call(S, u, h0, dinv)
    return h[:N]


# speedup vs baseline: 1.9516x; 1.2135x over previous
import functools

import jax
import jax.numpy as jnp
from jax import lax
from jax.experimental import pallas as pl
from jax.experimental.pallas import tpu as pltpu
from jax.experimental.pallas import tpu_sc as plsc

N = 10000
D = 128
K = 10
ALPHA = 0.1
EPS = 1e-5
E = 320000

NC = 2
NS = 16
NW = NC * NS

NPAD = 10112
RPT = NPAD // NS
CHUNK = 128
ECH = 79
EPT = ECH * CHUNK
EPAD = EPT * NW
NBUF = 2

def _edge_body(u_hbm, src_hbm, dst_hbm, out_hbm, s_sh, src_v, dst_b, rowbuf,
               zbuf, dsem, g0, g1):
    sems = [g0, g1]
    cid = lax.axis_index("c")
    sid = lax.axis_index("s")
    wid = cid * NS + sid
    for r in range(16):
        for j in range(D // 16):
            zbuf[r, pl.ds(j * 16, 16)] = jnp.zeros((16,), jnp.float32)
    zbase = sid * RPT
    def _z(i, c):
        pltpu.sync_copy(zbuf, s_sh.at[pl.ds(zbase + i * 16, 16)])
        return c
    lax.fori_loop(0, RPT // 16, _z, 0)
    pltpu.sync_copy(zbuf.at[pl.ds(0, 8)], s_sh.at[pl.ds(zbase + 624, 8)])
    pltpu.sync_copy(src_hbm.at[wid], src_v)
    for q in range(2):
        pltpu.async_copy(dst_hbm.at[wid].at[pl.ds(q * NBUF, NBUF)],
                         dst_b.at[q], dsem)
    plsc.subcore_barrier()
    for b in range(NBUF):
        pltpu.async_copy(u_hbm.at[src_v.at[b]], rowbuf.at[b], sems[b])
    def _outer(t, carry):
        q = lax.rem(t, 2)
        pltpu.make_async_copy(dst_hbm.at[wid].at[pl.ds(0, NBUF)],
                              dst_b.at[0], dsem).wait()
        for b in range(NBUF):
            c = t * NBUF + b
            pltpu.make_async_copy(u_hbm.at[src_v.at[b]], rowbuf.at[b],
                                  sems[b]).wait()
            pltpu.sync_copy(rowbuf.at[b], s_sh.at[dst_b.at[q].at[b]],
                            add=True)
            cp = c + NBUF
            @pl.when(cp < ECH)
            def _():
                pltpu.async_copy(u_hbm.at[src_v.at[cp]], rowbuf.at[b],
                                 sems[b])
        @pl.when(t + 2 < (ECH + NBUF - 1) // NBUF)
        def _():
            pltpu.async_copy(
                dst_hbm.at[wid].at[pl.ds((t + 2) * NBUF, NBUF)],
                dst_b.at[q], dsem)
        return carry
    lax.fori_loop(0, ECH // NBUF, _outer, 0)
    pltpu.make_async_copy(dst_hbm.at[wid].at[pl.ds(0, NBUF)],
                          dst_b.at[0], dsem).wait()
    pltpu.make_async_copy(u_hbm.at[src_v.at[0]], rowbuf.at[0],
                          sems[0]).wait()
    pltpu.sync_copy(rowbuf.at[0], s_sh.at[dst_b.at[1].at[0]], add=True)
    plsc.subcore_barrier()
    pltpu.sync_copy(s_sh.at[pl.ds(zbase, RPT)],
                    out_hbm.at[cid].at[pl.ds(zbase, RPT)])


def _edge_call(u, srcb, dstb):
    mesh = plsc.VectorSubcoreMesh(core_axis_name="c", subcore_axis_name="s")
    f = pl.kernel(
        _edge_body,
        out_type=jax.ShapeDtypeStruct((NC, NPAD, D), jnp.float32),
        mesh=mesh,
        scratch_types=[
            pltpu.VMEM_SHARED((NPAD, D), jnp.float32),
            pltpu.VMEM((ECH + 1, CHUNK), jnp.int32),
            pltpu.VMEM((2, NBUF, CHUNK), jnp.int32),
            pltpu.VMEM((NBUF, CHUNK, D), jnp.float32),
            pltpu.VMEM((16, D), jnp.float32),
            pltpu.SemaphoreType.DMA,
            pltpu.SemaphoreType.DMA,
            pltpu.SemaphoreType.DMA,
        ],
    )
    return f(u, srcb, dstb)


def _gelu(x):
    return 0.5 * x * (1.0 + lax.erf(x * 0.7071067811865476))


def _ln(h, g, b):
    mu = jnp.mean(h, axis=-1, keepdims=True)
    var = jnp.mean((h - mu) ** 2, axis=-1, keepdims=True)
    return (h - mu) * jax.lax.rsqrt(var + EPS) * g + b


def _mlp_body(x_ref, w0_ref, b0_ref, g0_ref, bt0_ref, w1_ref, b1_ref, g1_ref,
              bt1_ref, w2_ref, b2_ref, degp_ref, h0_ref, u0_ref, dinv_ref):
    x = x_ref[...]
    h = jnp.dot(x, w0_ref[...], preferred_element_type=jnp.float32) + b0_ref[...]
    h = _gelu(h)
    h = _ln(h, g0_ref[...], bt0_ref[...])
    h = jnp.dot(h, w1_ref[...], preferred_element_type=jnp.float32) + b1_ref[...]
    h = _gelu(h)
    h = _ln(h, g1_ref[...], bt1_ref[...])
    h = jnp.dot(h, w2_ref[...], preferred_element_type=jnp.float32) + b2_ref[...]
    degp = degp_ref[...]
    deg = degp[0, :, 0] + degp[1, :, 0] + 1.0
    dinv = jax.lax.rsqrt(deg)[:, None]
    rows = lax.broadcasted_iota(jnp.int32, (NPAD, 1), 0)
    h = jnp.where(rows < N, h, 0.0)
    h0_ref[...] = h
    u0_ref[...] = h * dinv
    dinv_ref[...] = jnp.broadcast_to(dinv, (NPAD, D))


def _mlp_call(xpad, W0, b0, g0, bt0, W1, b1, g1, bt1, W2, b2, degp):
    out = [jax.ShapeDtypeStruct((NPAD, D), jnp.float32)] * 3
    return pl.pallas_call(_mlp_body, out_shape=out)(
        xpad, W0, b0, g0, bt0, W1, b1, g1, bt1, W2, b2, degp)


def _upd_body(s_ref, u_ref, u0_ref, dinv_ref, out_ref):
    s = s_ref[0] + s_ref[1] + u_ref[...]
    di = dinv_ref[...]
    out_ref[...] = (1.0 - ALPHA) * di * di * s + ALPHA * u0_ref[...]


def _upd_call(S, u, u0, dinv):
    return pl.pallas_call(
        _upd_body, out_shape=jax.ShapeDtypeStruct((NPAD, D), jnp.float32)
    )(S, u, u0, dinv)


def _fin_body(s_ref, u_ref, h0_ref, dinv_ref, out_ref):
    s = s_ref[0] + s_ref[1] + u_ref[...]
    out_ref[...] = (1.0 - ALPHA) * dinv_ref[...] * s + ALPHA * h0_ref[...]


def _fin_call(S, u, h0, dinv):
    return pl.pallas_call(
        _fin_body, out_shape=jax.ShapeDtypeStruct((NPAD, D), jnp.float32)
    )(S, u, h0, dinv)


def kernel(x, edge_index, W0, b0, g0, bt0, W1, b1, g1, bt1, W2, b2):
    src = edge_index[0]
    dst = edge_index[1]
    fill = jnp.full((EPAD - E,), N, jnp.int32)
    srcb = jnp.concatenate([src, fill]).reshape(NW, ECH, CHUNK)
    srcb = jnp.pad(srcb, ((0, 0), (0, 1), (0, 0)), constant_values=N)
    dstb = jnp.concatenate([dst, fill]).reshape(NW, ECH, CHUNK)
    dstb = jnp.pad(dstb, ((0, 0), (0, 1), (0, 0)), constant_values=N)
    xpad = jnp.pad(x, ((0, NPAD - N), (0, 0)))

    ones = jnp.ones((NPAD, D), jnp.float32)
    degp = _edge_call(ones, srcb, dstb)
    h0, u0, dinv = _mlp_call(xpad, W0, b0, g0, bt0, W1, b1, g1, bt1, W2, b2,
                             degp)
    u = u0
    for k in range(K):
        S = _edge_call(u, srcb, dstb)
        if k < K - 1:
            u = _upd_call(S, u, u0, dinv)
        else:
            h = _fin_call(S, u, h0, dinv)
    return h[:N]
